# Initial kernel scaffold; baseline (speedup 1.0000x reference)
import functools

import jax
import jax.numpy as jnp
from jax import lax
from jax.experimental import pallas as pl
from jax.experimental.pallas import tpu as pltpu
from jax.experimental.pallas import tpu_sc as plsc

NUM_LAYERS = 10
NC = 2
NS = 16
NW = NC * NS
LANES = 16
EB = 80


def _zero_vmem_2d(ref, rows):
    z = jnp.zeros((LANES,), jnp.float32)

    def body(r, _):
        for k in range(128 // LANES):
            ref[r, pl.ds(k * LANES, LANES)] = z
        return 0

    lax.fori_loop(0, rows, body, 0)


def _zero_vmem_1d(ref, n):
    z = jnp.zeros((LANES,), jnp.float32)

    def body(r, _):
        ref[pl.ds(r * LANES, LANES)] = z
        return 0

    lax.fori_loop(0, n // LANES, body, 0)


def _deg_partial(dst, n_nodes):
    e = dst.shape[0]
    epw = e // NW
    iters = epw // EB
    mesh = plsc.VectorSubcoreMesh(core_axis_name="c", subcore_axis_name="s")
    zchunk = 2000
    ndrain = n_nodes // zchunk

    @functools.partial(
        pl.kernel,
        out_type=jax.ShapeDtypeStruct((NC, n_nodes), jnp.float32),
        mesh=mesh,
        scratch_types=[
            pltpu.VMEM_SHARED((n_nodes,), jnp.float32),
            pltpu.VMEM((EB,), jnp.int32),
            pltpu.VMEM((EB,), jnp.float32),
            pltpu.VMEM((zchunk,), jnp.float32),
        ],
    )
    def k(dst_hbm, deg_out, deg_sh, idx_v, ones_v, zbuf):
        c = lax.axis_index("c")
        s = lax.axis_index("s")
        wid = c * NS + s
        for j in range(EB // LANES):
            ones_v[pl.ds(j * LANES, LANES)] = jnp.ones((LANES,), jnp.float32)
        _zero_vmem_1d(zbuf, zchunk)

        @pl.when(s < ndrain)
        def _():
            pltpu.sync_copy(zbuf, deg_sh.at[pl.ds(s * zchunk, zchunk)])

        plsc.subcore_barrier()

        def body(i, _):
            base = wid * epw + i * EB
            pltpu.sync_copy(dst_hbm.at[pl.ds(base, EB)], idx_v)
            pltpu.sync_copy(ones_v, deg_sh.at[idx_v], add=True)
            return 0

        lax.fori_loop(0, iters, body, 0)
        plsc.subcore_barrier()

        @pl.when(s < ndrain)
        def _():
            pltpu.sync_copy(deg_sh.at[pl.ds(s * zchunk, zchunk)],
                            deg_out.at[c, pl.ds(s * zchunk, zchunk)])

    return k(dst)


def _prop_partial(outs, src, dst):
    n, d = outs.shape
    e = src.shape[0]
    epw = e // NW
    iters = epw // EB
    rpw = n // NS
    zrows = 125
    mesh = plsc.VectorSubcoreMesh(core_axis_name="c", subcore_axis_name="s")

    @functools.partial(
        pl.kernel,
        out_type=jax.ShapeDtypeStruct((NC, n, d), jnp.float32),
        mesh=mesh,
        scratch_types=[
            pltpu.VMEM_SHARED((n, d), jnp.float32),
            pltpu.VMEM((EB,), jnp.int32),
            pltpu.VMEM((EB,), jnp.int32),
            pltpu.VMEM((EB, d), jnp.float32),
            pltpu.VMEM((zrows, d), jnp.float32),
            pltpu.SemaphoreType.DMA,
        ],
    )
    def k(outs_hbm, src_hbm, dst_hbm, agg_out, agg_sh, idx_s, idx_d, rows_v,
          zbuf, sem):
        c = lax.axis_index("c")
        s = lax.axis_index("s")
        wid = c * NS + s
        _zero_vmem_2d(zbuf, zrows)
        for j in range(rpw // zrows):
            pltpu.sync_copy(zbuf, agg_sh.at[pl.ds(s * rpw + j * zrows, zrows)])
        plsc.subcore_barrier()

        def body(i, _):
            base = wid * epw + i * EB
            pltpu.sync_copy(src_hbm.at[pl.ds(base, EB)], idx_s)
            pltpu.sync_copy(dst_hbm.at[pl.ds(base, EB)], idx_d)
            pltpu.async_copy(outs_hbm.at[idx_s], rows_v, sem).wait()
            pltpu.sync_copy(rows_v, agg_sh.at[idx_d], add=True)
            return 0

        lax.fori_loop(0, iters, body, 0)
        plsc.subcore_barrier()
        pltpu.sync_copy(agg_sh.at[pl.ds(s * rpw, rpw)],
                        agg_out.at[c, pl.ds(s * rpw, rpw)])

    return k(outs, src, dst)


def _init_tc(labels2, maskf, protos, degt, alpha_arr):
    n = labels2.shape[0]
    c, d = protos.shape
    blk = 1000
    grid = n // blk

    def body(lab_ref, mask_ref, proto_ref, deg_ref, alpha_ref,
             res_ref, outs_ref, dinv_ref):
        alpha = alpha_ref[0, 0]
        lab = lab_ref[...]
        iota = lax.broadcasted_iota(jnp.int32, (blk, c), 1)
        oh = jnp.where(iota == lab, 1.0, 0.0).astype(jnp.float32)
        p = jnp.dot(oh, proto_ref[...], preferred_element_type=jnp.float32)
        p = p * mask_ref[...]
        deg = deg_ref[:, 0:1] + deg_ref[:, 1:2]
        dinv = jnp.where(deg > 0, lax.rsqrt(jnp.maximum(deg, 1e-12)), 0.0)
        res_ref[...] = (1.0 - alpha) * p
        outs_ref[...] = p * dinv
        dinv_ref[...] = dinv

    return pl.pallas_call(
        body,
        grid=(grid,),
        in_specs=[
            pl.BlockSpec((blk, 1), lambda i: (i, 0)),
            pl.BlockSpec((blk, 1), lambda i: (i, 0)),
            pl.BlockSpec((c, d), lambda i: (0, 0)),
            pl.BlockSpec((blk, 2), lambda i: (i, 0)),
            pl.BlockSpec((1, 1), lambda i: (0, 0)),
        ],
        out_specs=[
            pl.BlockSpec((blk, d), lambda i: (i, 0)),
            pl.BlockSpec((blk, d), lambda i: (i, 0)),
            pl.BlockSpec((blk, 1), lambda i: (i, 0)),
        ],
        out_shape=[
            jax.ShapeDtypeStruct((n, d), jnp.float32),
            jax.ShapeDtypeStruct((n, d), jnp.float32),
            jax.ShapeDtypeStruct((n, 1), jnp.float32),
        ],
    )(labels2, maskf, protos, degt, alpha_arr)


def _combine_tc(aggp, res, dinv, alpha_arr):
    _, n, d = aggp.shape
    blk = 1000
    grid = n // blk

    def body(agg_ref, res_ref, dinv_ref, alpha_ref, out_ref, outs_ref):
        alpha = alpha_ref[0, 0]
        agg = agg_ref[0] + agg_ref[1]
        dinv = dinv_ref[...]
        t = alpha * (agg * dinv) + res_ref[...]
        t = jnp.clip(t, 0.0, 1.0)
        out_ref[...] = t
        outs_ref[...] = t * dinv

    return pl.pallas_call(
        body,
        grid=(grid,),
        in_specs=[
            pl.BlockSpec((2, blk, d), lambda i: (0, i, 0)),
            pl.BlockSpec((blk, d), lambda i: (i, 0)),
            pl.BlockSpec((blk, 1), lambda i: (i, 0)),
            pl.BlockSpec((1, 1), lambda i: (0, 0)),
        ],
        out_specs=[
            pl.BlockSpec((blk, d), lambda i: (i, 0)),
            pl.BlockSpec((blk, d), lambda i: (i, 0)),
        ],
        out_shape=[
            jax.ShapeDtypeStruct((n, d), jnp.float32),
            jax.ShapeDtypeStruct((n, d), jnp.float32),
        ],
    )(aggp, res, dinv, alpha_arr)


def kernel(train_mask, global_protos, labels_all, edge_index, alpha):
    n = train_mask.shape[0]
    src = edge_index[0]
    dst = edge_index[1]
    maskf = train_mask.astype(jnp.float32).reshape(n, 1)
    labels2 = labels_all.astype(jnp.int32).reshape(n, 1)
    alpha_arr = jnp.asarray(alpha, jnp.float32).reshape(1, 1)

    degp = _deg_partial(dst, n)
    degt = degp.T
    res, outs, dinv = _init_tc(labels2, maskf, global_protos, degt, alpha_arr)
    out = None
    for _ in range(NUM_LAYERS):
        aggp = _prop_partial(outs, src, dst)
        out, outs = _combine_tc(aggp, res, dinv, alpha_arr)
    return out


# speedup vs baseline: 6.2679x; 6.2679x over previous
import functools

import jax
import jax.numpy as jnp
from jax import lax
from jax.experimental import pallas as pl
from jax.experimental.pallas import tpu as pltpu
from jax.experimental.pallas import tpu_sc as plsc

NUM_LAYERS = 10
NC = 2
NS = 16
NW = NC * NS
LANES = 16
EB = 80


def _zero_vmem_2d(ref, rows):
    z = jnp.zeros((LANES,), jnp.float32)

    def body(r, _):
        for k in range(128 // LANES):
            ref[r, pl.ds(k * LANES, LANES)] = z
        return 0

    lax.fori_loop(0, rows, body, 0)


def _zero_vmem_1d(ref, n):
    z = jnp.zeros((LANES,), jnp.float32)

    def body(r, _):
        ref[pl.ds(r * LANES, LANES)] = z
        return 0

    lax.fori_loop(0, n // LANES, body, 0)


def _deg_partial(dst, n_nodes):
    e = dst.shape[0]
    epw = e // NW
    iters = epw // EB
    mesh = plsc.VectorSubcoreMesh(core_axis_name="c", subcore_axis_name="s")
    zchunk = 2000
    ndrain = n_nodes // zchunk

    @functools.partial(
        pl.kernel,
        out_type=jax.ShapeDtypeStruct((NC * n_nodes,), jnp.float32),
        mesh=mesh,
        scratch_types=[
            pltpu.VMEM_SHARED((n_nodes,), jnp.float32),
            pltpu.VMEM((EB,), jnp.int32),
            pltpu.VMEM((EB,), jnp.float32),
            pltpu.VMEM((zchunk,), jnp.float32),
        ],
    )
    def k(dst_hbm, deg_out, deg_sh, idx_v, ones_v, zbuf):
        c = lax.axis_index("c")
        s = lax.axis_index("s")
        wid = c * NS + s
        for j in range(EB // LANES):
            ones_v[pl.ds(j * LANES, LANES)] = jnp.ones((LANES,), jnp.float32)
        _zero_vmem_1d(zbuf, zchunk)

        @pl.when(s < ndrain)
        def _():
            pltpu.sync_copy(zbuf, deg_sh.at[pl.ds(s * zchunk, zchunk)])

        plsc.subcore_barrier()

        def body(i, _):
            base = wid * epw + i * EB
            pltpu.sync_copy(dst_hbm.at[pl.ds(base, EB)], idx_v)
            pltpu.sync_copy(ones_v, deg_sh.at[idx_v], add=True)
            return 0

        lax.fori_loop(0, iters, body, 0)
        plsc.subcore_barrier()

        @pl.when(s < ndrain)
        def _():
            pltpu.sync_copy(deg_sh.at[pl.ds(s * zchunk, zchunk)], zbuf)
            pltpu.sync_copy(zbuf,
                            deg_out.at[pl.ds(c * n_nodes + s * zchunk, zchunk)])

    return k(dst).reshape(NC, n_nodes)


def _prop_partial(outs, src, dst):
    n, d = outs.shape
    e = src.shape[0]
    epw = e // NW
    iters = epw // EB
    ndrain = 10
    slab = n // ndrain
    zrows = 200
    mesh = plsc.VectorSubcoreMesh(core_axis_name="c", subcore_axis_name="s")

    @functools.partial(
        pl.kernel,
        out_type=jax.ShapeDtypeStruct((NC, n, d), jnp.float32),
        mesh=mesh,
        scratch_types=[
            pltpu.VMEM_SHARED((n, d), jnp.float32),
            pltpu.VMEM((EB,), jnp.int32),
            pltpu.VMEM((EB,), jnp.int32),
            pltpu.VMEM((EB, d), jnp.float32),
            pltpu.VMEM((zrows, d), jnp.float32),
            pltpu.SemaphoreType.DMA,
        ],
    )
    def k(outs_hbm, src_hbm, dst_hbm, agg_out, agg_sh, idx_s, idx_d, rows_v,
          zbuf, sem):
        c = lax.axis_index("c")
        s = lax.axis_index("s")
        wid = c * NS + s
        _zero_vmem_2d(zbuf, zrows)

        @pl.when(s < ndrain)
        def _():
            for j in range(slab // zrows):
                pltpu.sync_copy(zbuf,
                                agg_sh.at[pl.ds(s * slab + j * zrows, zrows)])

        plsc.subcore_barrier()

        def body(i, _):
            base = wid * epw + i * EB
            pltpu.sync_copy(src_hbm.at[pl.ds(base, EB)], idx_s)
            pltpu.sync_copy(dst_hbm.at[pl.ds(base, EB)], idx_d)
            pltpu.async_copy(outs_hbm.at[idx_s], rows_v, sem).wait()
            pltpu.sync_copy(rows_v, agg_sh.at[idx_d], add=True)
            return 0

        lax.fori_loop(0, iters, body, 0)
        plsc.subcore_barrier()

        @pl.when(s < ndrain)
        def _():
            for j in range(slab // zrows):
                base = s * slab + j * zrows
                pltpu.sync_copy(agg_sh.at[pl.ds(base, zrows)], zbuf)
                pltpu.sync_copy(zbuf, agg_out.at[c, pl.ds(base, zrows)])

    return k(outs, src, dst)


def _init_tc(labels2, maskf, protos, degt, alpha_arr):
    n = labels2.shape[0]
    c, d = protos.shape
    blk = 1000
    grid = n // blk

    def body(lab_ref, mask_ref, proto_ref, deg_ref, alpha_ref,
             res_ref, outs_ref, dinv_ref):
        alpha = alpha_ref[0, 0]
        lab = lab_ref[...]
        iota = lax.broadcasted_iota(jnp.int32, (blk, c), 1)
        oh = jnp.where(iota == lab, 1.0, 0.0).astype(jnp.float32)
        p = jnp.dot(oh, proto_ref[...], preferred_element_type=jnp.float32)
        p = p * mask_ref[...]
        deg = deg_ref[:, 0:1] + deg_ref[:, 1:2]
        dinv = jnp.where(deg > 0, lax.rsqrt(jnp.maximum(deg, 1e-12)), 0.0)
        res_ref[...] = (1.0 - alpha) * p
        outs_ref[...] = p * dinv
        dinv_ref[...] = dinv

    return pl.pallas_call(
        body,
        grid=(grid,),
        in_specs=[
            pl.BlockSpec((blk, 1), lambda i: (i, 0)),
            pl.BlockSpec((blk, 1), lambda i: (i, 0)),
            pl.BlockSpec((c, d), lambda i: (0, 0)),
            pl.BlockSpec((blk, 2), lambda i: (i, 0)),
            pl.BlockSpec((1, 1), lambda i: (0, 0)),
        ],
        out_specs=[
            pl.BlockSpec((blk, d), lambda i: (i, 0)),
            pl.BlockSpec((blk, d), lambda i: (i, 0)),
            pl.BlockSpec((blk, 1), lambda i: (i, 0)),
        ],
        out_shape=[
            jax.ShapeDtypeStruct((n, d), jnp.float32),
            jax.ShapeDtypeStruct((n, d), jnp.float32),
            jax.ShapeDtypeStruct((n, 1), jnp.float32),
        ],
    )(labels2, maskf, protos, degt, alpha_arr)


def _combine_tc(aggp, res, dinv, alpha_arr):
    _, n, d = aggp.shape
    blk = 1000
    grid = n // blk

    def body(agg_ref, res_ref, dinv_ref, alpha_ref, out_ref, outs_ref):
        alpha = alpha_ref[0, 0]
        agg = agg_ref[0] + agg_ref[1]
        dinv = dinv_ref[...]
        t = alpha * (agg * dinv) + res_ref[...]
        t = jnp.clip(t, 0.0, 1.0)
        out_ref[...] = t
        outs_ref[...] = t * dinv

    return pl.pallas_call(
        body,
        grid=(grid,),
        in_specs=[
            pl.BlockSpec((2, blk, d), lambda i: (0, i, 0)),
            pl.BlockSpec((blk, d), lambda i: (i, 0)),
            pl.BlockSpec((blk, 1), lambda i: (i, 0)),
            pl.BlockSpec((1, 1), lambda i: (0, 0)),
        ],
        out_specs=[
            pl.BlockSpec((blk, d), lambda i: (i, 0)),
            pl.BlockSpec((blk, d), lambda i: (i, 0)),
        ],
        out_shape=[
            jax.ShapeDtypeStruct((n, d), jnp.float32),
            jax.ShapeDtypeStruct((n, d), jnp.float32),
        ],
    )(aggp, res, dinv, alpha_arr)


def kernel(train_mask, global_protos, labels_all, edge_index, alpha):
    n = train_mask.shape[0]
    src = edge_index[0]
    dst = edge_index[1]
    maskf = train_mask.astype(jnp.float32).reshape(n, 1)
    labels2 = labels_all.astype(jnp.int32).reshape(n, 1)
    alpha_arr = jnp.asarray(alpha, jnp.float32).reshape(1, 1)

    degp = _deg_partial(dst, n)
    degt = degp.T
    res, outs, dinv = _init_tc(labels2, maskf, global_protos, degt, alpha_arr)
    out = None
    for _ in range(NUM_LAYERS):
        aggp = _prop_partial(outs, src, dst)
        out, outs = _combine_tc(aggp, res, dinv, alpha_arr)
    return out


# speedup vs baseline: 14.1979x; 2.2652x over previous
import functools

import jax
import jax.numpy as jnp
from jax import lax
from jax.experimental import pallas as pl
from jax.experimental.pallas import tpu as pltpu
from jax.experimental.pallas import tpu_sc as plsc

NUM_LAYERS = 10
NC = 2
NS = 16
NW = NC * NS
LANES = 16
EB = 80


def _zero_vmem_2d(ref, rows):
    z = jnp.zeros((LANES,), jnp.float32)

    def body(r, _):
        for k in range(128 // LANES):
            ref[r, pl.ds(k * LANES, LANES)] = z
        return 0

    lax.fori_loop(0, rows, body, 0)


def _zero_vmem_1d(ref, n):
    z = jnp.zeros((LANES,), jnp.float32)

    def body(r, _):
        ref[pl.ds(r * LANES, LANES)] = z
        return 0

    lax.fori_loop(0, n // LANES, body, 0)


def _deg_partial(dst, n_nodes):
    e = dst.shape[0]
    epw = e // NW
    iters = epw // EB
    mesh = plsc.VectorSubcoreMesh(core_axis_name="c", subcore_axis_name="s")
    zchunk = 2000
    ndrain = n_nodes // zchunk

    @functools.partial(
        pl.kernel,
        out_type=jax.ShapeDtypeStruct((NC * n_nodes,), jnp.float32),
        mesh=mesh,
        scratch_types=[
            pltpu.VMEM_SHARED((n_nodes,), jnp.float32),
            pltpu.VMEM((EB,), jnp.int32),
            pltpu.VMEM((EB,), jnp.float32),
            pltpu.VMEM((zchunk,), jnp.float32),
        ],
    )
    def k(dst_hbm, deg_out, deg_sh, idx_v, ones_v, zbuf):
        c = lax.axis_index("c")
        s = lax.axis_index("s")
        wid = c * NS + s
        for j in range(EB // LANES):
            ones_v[pl.ds(j * LANES, LANES)] = jnp.ones((LANES,), jnp.float32)
        _zero_vmem_1d(zbuf, zchunk)

        @pl.when(s < ndrain)
        def _():
            pltpu.sync_copy(zbuf, deg_sh.at[pl.ds(s * zchunk, zchunk)])

        plsc.subcore_barrier()

        def body(i, _):
            base = wid * epw + i * EB
            pltpu.sync_copy(dst_hbm.at[pl.ds(base, EB)], idx_v)
            pltpu.sync_copy(ones_v, deg_sh.at[idx_v], add=True)
            return 0

        lax.fori_loop(0, iters, body, 0)
        plsc.subcore_barrier()

        @pl.when(s < ndrain)
        def _():
            pltpu.sync_copy(deg_sh.at[pl.ds(s * zchunk, zchunk)], zbuf)
            pltpu.sync_copy(zbuf,
                            deg_out.at[pl.ds(c * n_nodes + s * zchunk, zchunk)])

    return k(dst).reshape(NC, n_nodes)


def _prop_partial(outs, src, dst):
    n, d = outs.shape
    e = src.shape[0]
    epw = e // NW
    iters = epw // EB
    ndrain = 10
    slab = n // ndrain
    zrows = 40
    mesh = plsc.VectorSubcoreMesh(core_axis_name="c", subcore_axis_name="s")

    @functools.partial(
        pl.kernel,
        out_type=jax.ShapeDtypeStruct((NC, n, d), jnp.float32),
        mesh=mesh,
        scratch_types=[
            pltpu.VMEM_SHARED((n, d), jnp.float32),
            pltpu.VMEM((epw,), jnp.int32),
            pltpu.VMEM((EB,), jnp.int32),
            pltpu.VMEM((EB,), jnp.int32),
            pltpu.VMEM((EB, d), jnp.float32),
            pltpu.VMEM((EB, d), jnp.float32),
            pltpu.VMEM((zrows, d), jnp.float32),
            pltpu.SemaphoreType.DMA,
            pltpu.SemaphoreType.DMA,
            pltpu.SemaphoreType.DMA,
            pltpu.SemaphoreType.DMA,
        ],
    )
    def k(outs_hbm, src_hbm, dst_hbm, agg_out, agg_sh, idx_sf,
          d0, d1, rows0, rows1, zbuf, sem0, sem1, semd0, semd1):
        c = lax.axis_index("c")
        s = lax.axis_index("s")
        wid = c * NS + s
        pltpu.sync_copy(src_hbm.at[pl.ds(wid * epw, epw)], idx_sf)

        def gather(i, rows, sem):
            pltpu.async_copy(outs_hbm.at[idx_sf.at[pl.ds(i * EB, EB)]],
                             rows, sem)

        def wait(rows, sem):
            pltpu.make_async_copy(outs_hbm.at[idx_sf.at[pl.ds(0, EB)]],
                                  rows, sem).wait()

        def didx(i, dbuf, semd):
            pltpu.async_copy(dst_hbm.at[pl.ds(wid * epw + i * EB, EB)],
                             dbuf, semd)

        def dwait(dbuf, semd):
            pltpu.make_async_copy(dst_hbm.at[pl.ds(0, EB)], dbuf, semd).wait()

        gather(0, rows0, sem0)
        didx(0, d0, semd0)
        _zero_vmem_2d(zbuf, zrows)

        @pl.when(s < ndrain)
        def _():
            for j in range(slab // zrows):
                pltpu.sync_copy(zbuf,
                                agg_sh.at[pl.ds(s * slab + j * zrows, zrows)])

        plsc.subcore_barrier()

        def pair(p, _):
            i0 = 2 * p
            gather(i0 + 1, rows1, sem1)
            didx(i0 + 1, d1, semd1)
            wait(rows0, sem0)
            dwait(d0, semd0)
            pltpu.sync_copy(rows0, agg_sh.at[d0], add=True)
            gather(i0 + 2, rows0, sem0)
            didx(i0 + 2, d0, semd0)
            wait(rows1, sem1)
            dwait(d1, semd1)
            pltpu.sync_copy(rows1, agg_sh.at[d1], add=True)
            return 0

        lax.fori_loop(0, iters // 2, pair, 0)
        wait(rows0, sem0)
        dwait(d0, semd0)
        pltpu.sync_copy(rows0, agg_sh.at[d0], add=True)
        plsc.subcore_barrier()

        @pl.when(s < ndrain)
        def _():
            for j in range(slab // zrows):
                base = s * slab + j * zrows
                pltpu.sync_copy(agg_sh.at[pl.ds(base, zrows)], zbuf)
                pltpu.sync_copy(zbuf, agg_out.at[c, pl.ds(base, zrows)])

    return k(outs, src, dst)


def _init_tc(labels2, maskf, protos, degt, alpha_arr):
    n = labels2.shape[0]
    c, d = protos.shape
    blk = 1000
    grid = n // blk

    def body(lab_ref, mask_ref, proto_ref, deg_ref, alpha_ref,
             res_ref, outs_ref, dinv_ref):
        alpha = alpha_ref[0, 0]
        lab = lab_ref[...]
        iota = lax.broadcasted_iota(jnp.int32, (blk, c), 1)
        oh = jnp.where(iota == lab, 1.0, 0.0).astype(jnp.float32)
        p = jnp.dot(oh, proto_ref[...], preferred_element_type=jnp.float32)
        p = p * mask_ref[...]
        deg = deg_ref[:, 0:1] + deg_ref[:, 1:2]
        dinv = jnp.where(deg > 0, lax.rsqrt(jnp.maximum(deg, 1e-12)), 0.0)
        res_ref[...] = (1.0 - alpha) * p
        outs_ref[...] = p * dinv
        dinv_ref[...] = dinv

    return pl.pallas_call(
        body,
        grid=(grid,),
        in_specs=[
            pl.BlockSpec((blk, 1), lambda i: (i, 0)),
            pl.BlockSpec((blk, 1), lambda i: (i, 0)),
            pl.BlockSpec((c, d), lambda i: (0, 0)),
            pl.BlockSpec((blk, 2), lambda i: (i, 0)),
            pl.BlockSpec((1, 1), lambda i: (0, 0)),
        ],
        out_specs=[
            pl.BlockSpec((blk, d), lambda i: (i, 0)),
            pl.BlockSpec((blk, d), lambda i: (i, 0)),
            pl.BlockSpec((blk, 1), lambda i: (i, 0)),
        ],
        out_shape=[
            jax.ShapeDtypeStruct((n, d), jnp.float32),
            jax.ShapeDtypeStruct((n, d), jnp.float32),
            jax.ShapeDtypeStruct((n, 1), jnp.float32),
        ],
    )(labels2, maskf, protos, degt, alpha_arr)


def _combine_tc(aggp, res, dinv, alpha_arr):
    _, n, d = aggp.shape
    blk = 1000
    grid = n // blk

    def body(agg_ref, res_ref, dinv_ref, alpha_ref, out_ref, outs_ref):
        alpha = alpha_ref[0, 0]
        agg = agg_ref[0] + agg_ref[1]
        dinv = dinv_ref[...]
        t = alpha * (agg * dinv) + res_ref[...]
        t = jnp.clip(t, 0.0, 1.0)
        out_ref[...] = t
        outs_ref[...] = t * dinv

    return pl.pallas_call(
        body,
        grid=(grid,),
        in_specs=[
            pl.BlockSpec((2, blk, d), lambda i: (0, i, 0)),
            pl.BlockSpec((blk, d), lambda i: (i, 0)),
            pl.BlockSpec((blk, 1), lambda i: (i, 0)),
            pl.BlockSpec((1, 1), lambda i: (0, 0)),
        ],
        out_specs=[
            pl.BlockSpec((blk, d), lambda i: (i, 0)),
            pl.BlockSpec((blk, d), lambda i: (i, 0)),
        ],
        out_shape=[
            jax.ShapeDtypeStruct((n, d), jnp.float32),
            jax.ShapeDtypeStruct((n, d), jnp.float32),
        ],
    )(aggp, res, dinv, alpha_arr)


def kernel(train_mask, global_protos, labels_all, edge_index, alpha):
    n = train_mask.shape[0]
    src = edge_index[0]
    dst = edge_index[1]
    maskf = train_mask.astype(jnp.float32).reshape(n, 1)
    labels2 = labels_all.astype(jnp.int32).reshape(n, 1)
    alpha_arr = jnp.asarray(alpha, jnp.float32).reshape(1, 1)

    degp = _deg_partial(dst, n)
    degt = degp.T
    res, outs, dinv = _init_tc(labels2, maskf, global_protos, degt, alpha_arr)
    out = None
    for _ in range(NUM_LAYERS):
        aggp = _prop_partial(outs, src, dst)
        out, outs = _combine_tc(aggp, res, dinv, alpha_arr)
    return out
